# Initial kernel scaffold; baseline (speedup 1.0000x reference)
import jax
import jax.numpy as jnp
from jax.experimental import pallas as pl


def kernel(samples, normals, points):
    raise NotImplementedError("write your pallas kernel here")


# speedup vs baseline: 2.1127x; 2.1127x over previous
import functools

import jax
import jax.numpy as jnp
from jax import lax
from jax.experimental import pallas as pl
from jax.experimental.pallas import tpu as pltpu
from jax.experimental.pallas import tpu_sc as plsc

N_PTS = 8192
N_SMP = 16384
W = 16
BQ = 256
N_CORES = 2
N_SUB = 16
HALF = N_SMP // N_CORES
GROUPS = HALF // 16
MAP = (0, 1, 2, 3, 1, 4, 5, 6, 2, 5, 7, 8, 3, 6, 8, 9)


def _dist_vals_body(s_ref, n_ref, p_ref, idx_ref, vals_ref):
    s = s_ref[...]
    d2 = (s[:, 0:1] - p_ref[0:1, :]) ** 2
    d2 = d2 + (s[:, 1:2] - p_ref[1:2, :]) ** 2
    d2 = d2 + (s[:, 2:3] - p_ref[2:3, :]) ** 2
    idx_ref[0, 0, :] = jnp.argmin(d2, axis=1).astype(jnp.int32)

    n = n_ref[...]
    d = -jnp.sum(n * s, axis=1, keepdims=True)
    n3 = n[:, 0:3]
    ps = jnp.concatenate([n3, d], axis=1)
    one = jnp.ones((BQ, 1), jnp.float32)
    zero2 = jnp.zeros((BQ, 2), jnp.float32)
    a = jnp.concatenate(
        [jnp.broadcast_to(ps[:, 0:1], (BQ, 4)),
         jnp.broadcast_to(ps[:, 1:2], (BQ, 3)),
         jnp.broadcast_to(ps[:, 2:3], (BQ, 2)),
         ps[:, 3:4], n3, one, zero2], axis=1)
    b = jnp.concatenate(
        [ps[:, 0:4], ps[:, 1:4], ps[:, 2:4], ps[:, 3:4],
         jnp.ones((BQ, 4), jnp.float32), zero2], axis=1)
    vals_ref[...] = (a * b).T


def _make_dist_vals(interpret=False):
    return pl.pallas_call(
        _dist_vals_body,
        grid=(N_SMP // BQ,),
        in_specs=[
            pl.BlockSpec((BQ, 8), lambda i: (i, 0)),
            pl.BlockSpec((BQ, 8), lambda i: (i, 0)),
            pl.BlockSpec((8, N_PTS), lambda i: (0, 0)),
        ],
        out_specs=[
            pl.BlockSpec((1, 1, BQ), lambda i: (i, 0, 0)),
            pl.BlockSpec((W, BQ), lambda i: (0, i)),
        ],
        out_shape=[
            jax.ShapeDtypeStruct((N_SMP // BQ, 1, BQ), jnp.int32),
            jax.ShapeDtypeStruct((W, N_SMP), jnp.float32),
        ],
        interpret=interpret,
    )


def _sc_scatter_body(vals_hbm, idx_hbm, zeros_hbm, out_hbm, idx_v, col_v, acc):
    cid = lax.axis_index("c")
    sid = lax.axis_index("s")
    pltpu.sync_copy(zeros_hbm, acc)
    pltpu.sync_copy(idx_hbm.at[pl.ds(cid * HALF, HALF)], idx_v)
    pltpu.sync_copy(vals_hbm.at[sid, pl.ds(cid * HALF, HALF)], col_v)

    def body(g, carry):
        iv = idx_v[pl.ds(g * 16, 16)]
        vv = col_v[pl.ds(g * 16, 16)]
        plsc.addupdate_scatter(acc, [iv], vv)
        return carry

    lax.fori_loop(0, GROUPS, body, 0)
    pltpu.sync_copy(acc, out_hbm.at[cid, sid])


@functools.cache
def _make_sc_scatter():
    return functools.partial(
        pl.kernel,
        out_type=jax.ShapeDtypeStruct((N_CORES, W, N_PTS), jnp.float32),
        mesh=plsc.VectorSubcoreMesh(core_axis_name="c", subcore_axis_name="s"),
        compiler_params=pltpu.CompilerParams(needs_layout_passes=False),
        scratch_types=[
            pltpu.VMEM((HALF,), jnp.int32),
            pltpu.VMEM((HALF,), jnp.float32),
            pltpu.VMEM((N_PTS,), jnp.float32),
        ],
    )(_sc_scatter_body)


def _fin_body(p_ref, q_ref, m_ref, nv_ref):
    p = p_ref[...]
    tot = p[0] + p[1]
    q_ref[...] = jnp.concatenate([tot[r:r + 1, :] for r in MAP], axis=0)
    cnt = tot[13:14, :]
    m_ref[...] = tot[10:13, :] / jnp.maximum(cnt, 1.0)
    nv_ref[...] = (cnt > 0.0).astype(jnp.int32)


def _make_fin(interpret=False):
    return pl.pallas_call(
        _fin_body,
        out_shape=[
            jax.ShapeDtypeStruct((16, N_PTS), jnp.float32),
            jax.ShapeDtypeStruct((3, N_PTS), jnp.float32),
            jax.ShapeDtypeStruct((1, N_PTS), jnp.int32),
        ],
        interpret=interpret,
    )


def kernel(samples, normals, points):
    s8 = jnp.zeros((N_SMP, 8), jnp.float32).at[:, 0:3].set(samples)
    n8 = jnp.zeros((N_SMP, 8), jnp.float32).at[:, 0:3].set(normals)
    pt = jnp.zeros((8, N_PTS), jnp.float32).at[0:3, :].set(points.T)
    idx3, vals_t = _make_dist_vals()(s8, n8, pt)
    idx = idx3.reshape(N_SMP)
    zeros = jnp.zeros((N_PTS,), jnp.float32)
    part = _make_sc_scatter()(vals_t, idx, zeros)
    q_t, m_t, nv_t = _make_fin()(part)
    return (q_t.T.reshape(N_PTS, 4, 4), m_t.T, nv_t.reshape(N_PTS).astype(bool))


# speedup vs baseline: 2.6967x; 1.2764x over previous
import functools

import jax
import jax.numpy as jnp
from jax import lax
from jax.experimental import pallas as pl
from jax.experimental.pallas import tpu as pltpu
from jax.experimental.pallas import tpu_sc as plsc

N_PTS = 8192
N_SMP = 16384
W = 16
BQ = 128
CP = 128
N_CORES = 2
N_SUB = 16
HALF = N_SMP // N_CORES
GROUPS = HALF // 16
MAP = (0, 1, 2, 3, 1, 4, 5, 6, 2, 5, 7, 8, 3, 6, 8, 9)


def _dist_vals_body(s_ref, n_ref, p_ref, idx_ref, vals_ref):
    s = s_ref[...]
    s0, s1, s2 = s[:, 0:1], s[:, 1:2], s[:, 2:3]
    run_val = jnp.full((BQ, CP), jnp.inf, jnp.float32)
    run_idx = jnp.zeros((BQ, CP), jnp.int32)
    lane = lax.broadcasted_iota(jnp.int32, (BQ, CP), 1)
    for c in range(N_PTS // CP):
        pc = pl.ds(c * CP, CP)
        d2 = (s0 - p_ref[0:1, pc]) ** 2
        d2 = d2 + (s1 - p_ref[1:2, pc]) ** 2
        d2 = d2 + (s2 - p_ref[2:3, pc]) ** 2
        upd = d2 < run_val
        run_val = jnp.where(upd, d2, run_val)
        run_idx = jnp.where(upd, lane + c * CP, run_idx)
    m = jnp.min(run_val, axis=1, keepdims=True)
    cand = jnp.where(run_val == m, run_idx, jnp.int32(0x7FFFFFFF))
    idx_ref[0, 0, :] = jnp.min(cand, axis=1)

    n = n_ref[...]
    d = -((n[:, 0:1] * s0 + n[:, 1:2] * s1) + n[:, 2:3] * s2)
    n3 = n
    ps = jnp.concatenate([n3, d], axis=1)
    one = jnp.ones((BQ, 1), jnp.float32)
    zero2 = jnp.zeros((BQ, 2), jnp.float32)
    a = jnp.concatenate(
        [jnp.broadcast_to(ps[:, 0:1], (BQ, 4)),
         jnp.broadcast_to(ps[:, 1:2], (BQ, 3)),
         jnp.broadcast_to(ps[:, 2:3], (BQ, 2)),
         ps[:, 3:4], n3, one, zero2], axis=1)
    b = jnp.concatenate(
        [ps[:, 0:4], ps[:, 1:4], ps[:, 2:4], ps[:, 3:4],
         jnp.ones((BQ, 4), jnp.float32), zero2], axis=1)
    vals_ref[...] = (a * b).T


def _make_dist_vals(interpret=False):
    return pl.pallas_call(
        _dist_vals_body,
        grid=(N_SMP // BQ,),
        in_specs=[
            pl.BlockSpec((BQ, 3), lambda i: (i, 0)),
            pl.BlockSpec((BQ, 3), lambda i: (i, 0)),
            pl.BlockSpec((8, N_PTS), lambda i: (0, 0)),
        ],
        out_specs=[
            pl.BlockSpec((1, 1, BQ), lambda i: (i, 0, 0)),
            pl.BlockSpec((W, BQ), lambda i: (0, i)),
        ],
        out_shape=[
            jax.ShapeDtypeStruct((N_SMP // BQ, 1, BQ), jnp.int32),
            jax.ShapeDtypeStruct((W, N_SMP), jnp.float32),
        ],
        interpret=interpret,
    )


def _sc_scatter_body(vals_hbm, idx_hbm, zeros_hbm, out_hbm, idx_v, col_v, acc):
    cid = lax.axis_index("c")
    sid = lax.axis_index("s")
    pltpu.sync_copy(zeros_hbm, acc)
    pltpu.sync_copy(idx_hbm.at[pl.ds(cid * HALF, HALF)], idx_v)
    pltpu.sync_copy(vals_hbm.at[sid, pl.ds(cid * HALF, HALF)], col_v)

    def body(g, carry):
        iv = idx_v[pl.ds(g * 16, 16)]
        vv = col_v[pl.ds(g * 16, 16)]
        plsc.addupdate_scatter(acc, [iv], vv)
        return carry

    lax.fori_loop(0, GROUPS, body, 0)
    pltpu.sync_copy(acc, out_hbm.at[cid, sid])


@functools.cache
def _make_sc_scatter():
    return functools.partial(
        pl.kernel,
        out_type=jax.ShapeDtypeStruct((N_CORES, W, N_PTS), jnp.float32),
        mesh=plsc.VectorSubcoreMesh(core_axis_name="c", subcore_axis_name="s"),
        compiler_params=pltpu.CompilerParams(needs_layout_passes=False),
        scratch_types=[
            pltpu.VMEM((HALF,), jnp.int32),
            pltpu.VMEM((HALF,), jnp.float32),
            pltpu.VMEM((N_PTS,), jnp.float32),
        ],
    )(_sc_scatter_body)


def _fin_body(p_ref, q_ref, m_ref, nv_ref):
    p = p_ref[...]
    tot = p[0] + p[1]
    q_ref[...] = jnp.concatenate([tot[r:r + 1, :] for r in MAP], axis=0)
    cnt = tot[13:14, :]
    m_ref[...] = tot[10:13, :] / jnp.maximum(cnt, 1.0)
    nv_ref[...] = (cnt > 0.0).astype(jnp.int32)


def _make_fin(interpret=False):
    return pl.pallas_call(
        _fin_body,
        out_shape=[
            jax.ShapeDtypeStruct((16, N_PTS), jnp.float32),
            jax.ShapeDtypeStruct((3, N_PTS), jnp.float32),
            jax.ShapeDtypeStruct((1, N_PTS), jnp.int32),
        ],
        interpret=interpret,
    )


def kernel(samples, normals, points):
    pt = jnp.zeros((8, N_PTS), jnp.float32).at[0:3, :].set(points.T)
    idx3, vals_t = _make_dist_vals()(samples, normals, pt)
    idx = idx3.reshape(N_SMP)
    zeros = jnp.zeros((N_PTS,), jnp.float32)
    part = _make_sc_scatter()(vals_t, idx, zeros)
    q_t, m_t, nv_t = _make_fin()(part)
    return (q_t.T.reshape(N_PTS, 4, 4), m_t.T, nv_t.reshape(N_PTS).astype(bool))


# speedup vs baseline: 2.7154x; 1.0069x over previous
import functools

import jax
import jax.numpy as jnp
from jax import lax
from jax.experimental import pallas as pl
from jax.experimental.pallas import tpu as pltpu
from jax.experimental.pallas import tpu_sc as plsc

N_PTS = 8192
N_SMP = 16384
W = 16
BQ = 128
CP = 256
N_CORES = 2
N_SUB = 16
HALF = N_SMP // N_CORES
GROUPS = HALF // 16
MAP = (0, 1, 2, 3, 1, 4, 5, 6, 2, 5, 7, 8, 3, 6, 8, 9)


def _dist_vals_body(s_ref, n_ref, p_ref, idx_ref, vals_ref):
    s = s_ref[...]
    s0, s1, s2 = s[:, 0:1], s[:, 1:2], s[:, 2:3]
    run_val = jnp.full((BQ, CP), jnp.inf, jnp.float32)
    run_chk = jnp.zeros((BQ, CP), jnp.int32)
    lane = lax.broadcasted_iota(jnp.int32, (BQ, CP), 1)
    for c in range(N_PTS // CP):
        pc = pl.ds(c * CP, CP)
        d2 = (s0 - p_ref[0:1, pc]) ** 2
        d2 = d2 + (s1 - p_ref[1:2, pc]) ** 2
        d2 = d2 + (s2 - p_ref[2:3, pc]) ** 2
        upd = d2 < run_val
        run_val = jnp.where(upd, d2, run_val)
        run_chk = jnp.where(upd, jnp.int32(c), run_chk)
    m = jnp.min(run_val, axis=1, keepdims=True)
    run_idx = run_chk * CP + lane
    cand = jnp.where(run_val == m, run_idx, jnp.int32(0x7FFFFFFF))
    idx_ref[0, 0, :] = jnp.min(cand, axis=1)

    n = n_ref[...]
    d = -((n[:, 0:1] * s0 + n[:, 1:2] * s1) + n[:, 2:3] * s2)
    n3 = n
    ps = jnp.concatenate([n3, d], axis=1)
    one = jnp.ones((BQ, 1), jnp.float32)
    zero2 = jnp.zeros((BQ, 2), jnp.float32)
    a = jnp.concatenate(
        [jnp.broadcast_to(ps[:, 0:1], (BQ, 4)),
         jnp.broadcast_to(ps[:, 1:2], (BQ, 3)),
         jnp.broadcast_to(ps[:, 2:3], (BQ, 2)),
         ps[:, 3:4], n3, one, zero2], axis=1)
    b = jnp.concatenate(
        [ps[:, 0:4], ps[:, 1:4], ps[:, 2:4], ps[:, 3:4],
         jnp.ones((BQ, 4), jnp.float32), zero2], axis=1)
    vals_ref[...] = (a * b).T


def _make_dist_vals(interpret=False):
    return pl.pallas_call(
        _dist_vals_body,
        grid=(N_SMP // BQ,),
        in_specs=[
            pl.BlockSpec((BQ, 3), lambda i: (i, 0)),
            pl.BlockSpec((BQ, 3), lambda i: (i, 0)),
            pl.BlockSpec((8, N_PTS), lambda i: (0, 0)),
        ],
        out_specs=[
            pl.BlockSpec((1, 1, BQ), lambda i: (i, 0, 0)),
            pl.BlockSpec((W, BQ), lambda i: (0, i)),
        ],
        out_shape=[
            jax.ShapeDtypeStruct((N_SMP // BQ, 1, BQ), jnp.int32),
            jax.ShapeDtypeStruct((W, N_SMP), jnp.float32),
        ],
        interpret=interpret,
    )


def _sc_scatter_body(vals_hbm, idx_hbm, zeros_hbm, out_hbm, idx_v, col_v, acc):
    cid = lax.axis_index("c")
    sid = lax.axis_index("s")
    pltpu.sync_copy(zeros_hbm, acc)
    pltpu.sync_copy(idx_hbm.at[pl.ds(cid * HALF, HALF)], idx_v)
    pltpu.sync_copy(vals_hbm.at[sid, pl.ds(cid * HALF, HALF)], col_v)

    def body(g, carry):
        iv = idx_v[pl.ds(g * 16, 16)]
        vv = col_v[pl.ds(g * 16, 16)]
        plsc.addupdate_scatter(acc, [iv], vv)
        return carry

    lax.fori_loop(0, GROUPS, body, 0)
    pltpu.sync_copy(acc, out_hbm.at[cid, sid])


@functools.cache
def _make_sc_scatter():
    return functools.partial(
        pl.kernel,
        out_type=jax.ShapeDtypeStruct((N_CORES, W, N_PTS), jnp.float32),
        mesh=plsc.VectorSubcoreMesh(core_axis_name="c", subcore_axis_name="s"),
        compiler_params=pltpu.CompilerParams(needs_layout_passes=False),
        scratch_types=[
            pltpu.VMEM((HALF,), jnp.int32),
            pltpu.VMEM((HALF,), jnp.float32),
            pltpu.VMEM((N_PTS,), jnp.float32),
        ],
    )(_sc_scatter_body)


def _fin_body(p_ref, q_ref, m_ref, nv_ref):
    p = p_ref[...]
    tot = p[0] + p[1]
    qt = jnp.concatenate([tot[r:r + 1, :] for r in MAP], axis=0)
    q_ref[...] = qt.T
    cnt = tot[13:14, :]
    m_ref[...] = (tot[10:13, :] / jnp.maximum(cnt, 1.0)).T
    nv_ref[...] = (cnt > 0.0).astype(jnp.int32)


def _make_fin(interpret=False):
    return pl.pallas_call(
        _fin_body,
        out_shape=[
            jax.ShapeDtypeStruct((N_PTS, 16), jnp.float32),
            jax.ShapeDtypeStruct((N_PTS, 3), jnp.float32),
            jax.ShapeDtypeStruct((1, N_PTS), jnp.int32),
        ],
        interpret=interpret,
    )


def kernel(samples, normals, points):
    pt = jnp.zeros((8, N_PTS), jnp.float32).at[0:3, :].set(points.T)
    idx3, vals_t = _make_dist_vals()(samples, normals, pt)
    idx = idx3.reshape(N_SMP)
    zeros = jnp.zeros((N_PTS,), jnp.float32)
    part = _make_sc_scatter()(vals_t, idx, zeros)
    q, mn, nv_t = _make_fin()(part)
    return (q.reshape(N_PTS, 4, 4), mn, nv_t.reshape(N_PTS).astype(bool))
